# baseline (speedup 1.0000x reference)
import functools

import jax
import jax.numpy as jnp
from jax import lax
from jax.experimental import pallas as pl
from jax.experimental.pallas import tpu as pltpu
from jax.experimental.pallas import tpu_sc as plsc

HIDDEN = 64
NQ = HIDDEN // 16
EPS = 1e-5
L = 16
NC, NS = 2, 16
NW = NC * NS
GCHUNK = 128
GROUP = 256
GPG = GROUP // GCHUNK
NBLK = GROUP // L


def _rsqrt(x):
    i = lax.bitcast_convert_type(x, jnp.int32)
    i = jnp.int32(0x5F3759DF) - (i >> 1)
    y = lax.bitcast_convert_type(i, jnp.float32)
    for _ in range(3):
        y = y * (1.5 - 0.5 * x * y * y)
    return y


def _lane(v, r):
    return v.at[jnp.full((L,), r, jnp.int32)].get(mode="promise_in_bounds")


def _make_sc_kernel(B):
    per_tile = B // NW
    ngroups = per_tile // GROUP
    nchunks = per_tile // GCHUNK
    mesh = plsc.VectorSubcoreMesh(
        core_axis_name="c", subcore_axis_name="s",
        num_cores=NC, num_subcores=NS)

    row_buf = lambda: pltpu.VMEM((GROUP, HIDDEN), jnp.float32)

    @functools.partial(
        pl.kernel,
        out_type=jax.ShapeDtypeStruct((B, HIDDEN), jnp.float32),
        mesh=mesh,
        scratch_types=[
            pltpu.VMEM((nchunks, GCHUNK), jnp.int32),
            row_buf(), row_buf(),
            row_buf(), row_buf(),
            pltpu.VMEM((HIDDEN,), jnp.float32),
            pltpu.VMEM((HIDDEN,), jnp.float32),
            pltpu.SemaphoreType.DMA, pltpu.SemaphoreType.DMA,
            pltpu.SemaphoreType.DMA, pltpu.SemaphoreType.DMA,
        ],
        compiler_params=pltpu.CompilerParams(
            needs_layout_passes=False, use_tc_tiling_on_sc=False),
    )
    def sc_kernel(idx_hbm, table_hbm, gamma_hbm, beta_hbm, out_hbm,
                  idx_v, ibuf0, ibuf1, obuf0, obuf1, gamma_v, beta_v,
                  gsem0, gsem1, osem0, osem1):
        ibuf = (ibuf0, ibuf1)
        obuf = (obuf0, obuf1)
        gsem = (gsem0, gsem1)
        osem = (osem0, osem1)
        wid = lax.axis_index("s") * NC + lax.axis_index("c")
        base = wid * per_tile

        pltpu.sync_copy(idx_hbm.at[pl.ds(wid * nchunks, nchunks)], idx_v)
        pltpu.sync_copy(gamma_hbm, gamma_v)
        pltpu.sync_copy(beta_hbm, beta_v)

        g4 = [gamma_v[pl.ds(q * L, L)] for q in range(NQ)]
        b4 = [beta_v[pl.ds(q * L, L)] for q in range(NQ)]
        iota = lax.iota(jnp.int32, L)

        def fire_gathers(g, b):
            for j in range(GPG):
                pltpu.async_copy(
                    table_hbm.at[idx_v.at[g * GPG + j]],
                    ibuf[b].at[pl.ds(j * GCHUNK, GCHUNK)],
                    gsem[b])

        def wait_gathers(g, b):
            for j in range(GPG):
                pltpu.make_async_copy(
                    table_hbm.at[idx_v.at[g * GPG + j]],
                    ibuf[b].at[pl.ds(j * GCHUNK, GCHUNK)],
                    gsem[b]).wait()

        def out_copy(g, b):
            return pltpu.make_async_copy(
                obuf[b], out_hbm.at[pl.ds(base + g * GROUP, GROUP)], osem[b])

        def compute_group(b):
            src, dst = ibuf[b], obuf[b]

            def block_body(blk, _):
                rows = blk * L + iota
                s = jnp.zeros((L,), jnp.float32)
                ss = jnp.zeros((L,), jnp.float32)
                for c in range(HIDDEN):
                    col = jnp.full((L,), c, jnp.int32)
                    v = plsc.load_gather(src, [rows, col])
                    s = s + v
                    ss = ss + v * v
                mean = s * (1.0 / HIDDEN)
                var = ss * (1.0 / HIDDEN) - mean * mean
                rstd = _rsqrt(var + EPS)
                nmr = -mean * rstd
                for r in range(L):
                    row = blk * L + r
                    a = _lane(rstd, r)
                    cshift = _lane(nmr, r)
                    for q in range(NQ):
                        x = src[row, pl.ds(q * L, L)]
                        dst[row, pl.ds(q * L, L)] = (
                            (x * a + cshift) * g4[q] + b4[q])
                return 0

            lax.fori_loop(0, NBLK, block_body, 0)

        fire_gathers(0, 0)
        fire_gathers(1, 1)

        def group_body(g, _):
            for phase in range(2):
                gg = g * 2 + phase
                pl.when(gg >= 2)(lambda: out_copy(gg - 2, phase).wait())
                wait_gathers(gg, phase)
                compute_group(phase)
                out_copy(gg, phase).start()
                pl.when(gg + 2 < ngroups)(
                    lambda: fire_gathers(gg + 2, phase))
            return 0

        lax.fori_loop(0, ngroups // 2, group_body, 0)
        out_copy(ngroups - 2, 0).wait()
        out_copy(ngroups - 1, 1).wait()

    return sc_kernel


def kernel(input_idx, table, ln_gamma, ln_beta):
    nb, nt = input_idx.shape
    B = nb * nt
    idx2d = input_idx.reshape(B // GCHUNK, GCHUNK).astype(jnp.int32)
    out = _make_sc_kernel(B)(idx2d, table, ln_gamma, ln_beta)
    return out.reshape(nb, nt, HIDDEN)


# speedup vs baseline: 1.2074x; 1.2074x over previous
import functools

import jax
import jax.numpy as jnp
from jax import lax
from jax.experimental import pallas as pl
from jax.experimental.pallas import tpu as pltpu
from jax.experimental.pallas import tpu_sc as plsc

HIDDEN = 64
NQ = HIDDEN // 16
EPS = 1e-5
L = 16
NC, NS = 2, 16
NW = NC * NS
GCHUNK = 128
GROUP = 256
GPG = GROUP // GCHUNK
NBLK = GROUP // L


def _rsqrt(x):
    i = lax.bitcast_convert_type(x, jnp.int32)
    i = jnp.int32(0x5F3759DF) - (i >> 1)
    y = lax.bitcast_convert_type(i, jnp.float32)
    for _ in range(3):
        y = y * (1.5 - 0.5 * x * y * y)
    return y


def _lane(v, r):
    return v.at[jnp.full((L,), r, jnp.int32)].get(mode="promise_in_bounds")


def _make_sc_kernel(B):
    per_tile = B // NW
    ngroups = per_tile // GROUP
    nchunks = per_tile // GCHUNK
    mesh = plsc.VectorSubcoreMesh(
        core_axis_name="c", subcore_axis_name="s",
        num_cores=NC, num_subcores=NS)

    row_buf = lambda: pltpu.VMEM((GROUP, HIDDEN), jnp.float32)

    @functools.partial(
        pl.kernel,
        out_type=jax.ShapeDtypeStruct((B, HIDDEN), jnp.float32),
        mesh=mesh,
        scratch_types=[
            pltpu.VMEM((nchunks, GCHUNK), jnp.int32),
            row_buf(), row_buf(),
            row_buf(), row_buf(),
            pltpu.VMEM((HIDDEN,), jnp.float32),
            pltpu.VMEM((HIDDEN,), jnp.float32),
            pltpu.SemaphoreType.DMA, pltpu.SemaphoreType.DMA,
            pltpu.SemaphoreType.DMA, pltpu.SemaphoreType.DMA,
        ],
        compiler_params=pltpu.CompilerParams(
            needs_layout_passes=False, use_tc_tiling_on_sc=False),
    )
    def sc_kernel(idx_hbm, table_hbm, gamma_hbm, beta_hbm, out_hbm,
                  idx_v, ibuf0, ibuf1, obuf0, obuf1, gamma_v, beta_v,
                  gsem0, gsem1, osem0, osem1):
        ibuf = (ibuf0, ibuf1)
        obuf = (obuf0, obuf1)
        gsem = (gsem0, gsem1)
        osem = (osem0, osem1)
        wid = lax.axis_index("s") * NC + lax.axis_index("c")
        base = wid * per_tile

        pltpu.sync_copy(idx_hbm.at[pl.ds(wid * nchunks, nchunks)], idx_v)
        pltpu.sync_copy(gamma_hbm, gamma_v)
        pltpu.sync_copy(beta_hbm, beta_v)

        g4 = [gamma_v[pl.ds(q * L, L)] for q in range(NQ)]
        b4 = [beta_v[pl.ds(q * L, L)] for q in range(NQ)]
        iota = lax.iota(jnp.int32, L)

        def fire_gathers(g, b):
            for j in range(GPG):
                pltpu.async_copy(
                    table_hbm.at[idx_v.at[g * GPG + j]],
                    ibuf[b].at[pl.ds(j * GCHUNK, GCHUNK)],
                    gsem[b])

        def wait_gathers(g, b):
            for j in range(GPG):
                pltpu.make_async_copy(
                    table_hbm.at[idx_v.at[g * GPG + j]],
                    ibuf[b].at[pl.ds(j * GCHUNK, GCHUNK)],
                    gsem[b]).wait()

        def out_copy(g, b):
            return pltpu.make_async_copy(
                obuf[b], out_hbm.at[pl.ds(base + g * GROUP, GROUP)], osem[b])

        def compute_group(b):
            src, dst = ibuf[b], obuf[b]

            def row_body(r, _):
                x = [src[r, pl.ds(q * L, L)] for q in range(NQ)]
                p = (x[0] + x[1]) + (x[2] + x[3])
                sq = (x[0] * x[0] + x[1] * x[1]) + (x[2] * x[2]
                                                    + x[3] * x[3])
                total = _lane(plsc.cumsum(p), L - 1)
                totsq = _lane(plsc.cumsum(sq), L - 1)
                mean = total * (1.0 / HIDDEN)
                var = totsq * (1.0 / HIDDEN) - mean * mean
                rstd = _rsqrt(var + EPS)
                nmr = -mean * rstd
                for q in range(NQ):
                    dst[r, pl.ds(q * L, L)] = (
                        (x[q] * rstd + nmr) * g4[q] + b4[q])
                return 0

            lax.fori_loop(0, GROUP, row_body, 0, unroll=4)

        fire_gathers(0, 0)
        fire_gathers(1, 1)

        def group_body(g, _):
            for phase in range(2):
                gg = g * 2 + phase
                pl.when(gg >= 2)(lambda: out_copy(gg - 2, phase).wait())
                wait_gathers(gg, phase)
                compute_group(phase)
                out_copy(gg, phase).start()
                pl.when(gg + 2 < ngroups)(
                    lambda: fire_gathers(gg + 2, phase))
            return 0

        lax.fori_loop(0, ngroups // 2, group_body, 0)
        out_copy(ngroups - 2, 0).wait()
        out_copy(ngroups - 1, 1).wait()

    return sc_kernel


def kernel(input_idx, table, ln_gamma, ln_beta):
    nb, nt = input_idx.shape
    B = nb * nt
    idx2d = input_idx.reshape(B // GCHUNK, GCHUNK).astype(jnp.int32)
    out = _make_sc_kernel(B)(idx2d, table, ln_gamma, ln_beta)
    return out.reshape(nb, nt, HIDDEN)


# speedup vs baseline: 2.2053x; 1.8265x over previous
import functools

import jax
import jax.numpy as jnp
from jax import lax
from jax.experimental import pallas as pl
from jax.experimental.pallas import tpu as pltpu
from jax.experimental.pallas import tpu_sc as plsc

HIDDEN = 64
NQ = HIDDEN // 16
EPS = 1e-5
L = 16
NC, NS = 2, 16
NW = NC * NS
GROUP = 128
PADW = 128


def _rsqrt(x):
    i = lax.bitcast_convert_type(x, jnp.int32)
    i = jnp.int32(0x5F3759DF) - (i >> 1)
    y = lax.bitcast_convert_type(i, jnp.float32)
    for _ in range(2):
        y = y * (1.5 - 0.5 * x * y * y)
    return y


def _bsum(v, iota):
    for sh in (8, 4, 2, 1):
        v = v + v.at[iota ^ sh].get(mode="promise_in_bounds")
    return v


def _make_sc_kernel(B):
    per_tile = B // NW
    ngroups = per_tile // GROUP
    mesh = plsc.VectorSubcoreMesh(
        core_axis_name="c", subcore_axis_name="s",
        num_cores=NC, num_subcores=NS)

    @functools.partial(
        pl.kernel,
        out_type=jax.ShapeDtypeStruct((B, HIDDEN), jnp.float32),
        mesh=mesh,
        scratch_types=[
            pltpu.VMEM((per_tile // GROUP, GROUP), jnp.int32),
            pltpu.VMEM((per_tile // GROUP, GROUP), jnp.int32),
            pltpu.VMEM((GROUP, PADW), jnp.float32),
            pltpu.VMEM((GROUP, PADW), jnp.float32),
            pltpu.VMEM((GROUP, HIDDEN), jnp.float32),
            pltpu.VMEM((GROUP, HIDDEN), jnp.float32),
            pltpu.VMEM((HIDDEN,), jnp.float32),
            pltpu.VMEM((HIDDEN,), jnp.float32),
            pltpu.SemaphoreType.DMA, pltpu.SemaphoreType.DMA,
            pltpu.SemaphoreType.DMA, pltpu.SemaphoreType.DMA,
        ],
        compiler_params=pltpu.CompilerParams(
            needs_layout_passes=False, use_tc_tiling_on_sc=True),
    )
    def sc_kernel(pair_hbm, par_hbm, table_hbm, gamma_hbm, beta_hbm,
                  out_hbm, pair_v, par_v, ibuf0, ibuf1, obuf0, obuf1,
                  gamma_v, beta_v, gsem0, gsem1, osem0, osem1):
        ibuf = (ibuf0, ibuf1)
        obuf = (obuf0, obuf1)
        gsem = (gsem0, gsem1)
        osem = (osem0, osem1)
        wid = lax.axis_index("s") * NC + lax.axis_index("c")
        base = wid * per_tile

        pltpu.sync_copy(pair_hbm.at[pl.ds(wid * ngroups, ngroups)], pair_v)
        pltpu.sync_copy(par_hbm.at[pl.ds(wid * ngroups, ngroups)], par_v)
        pltpu.sync_copy(gamma_hbm, gamma_v)
        pltpu.sync_copy(beta_hbm, beta_v)

        g4 = [gamma_v[pl.ds(q * L, L)] for q in range(NQ)]
        b4 = [beta_v[pl.ds(q * L, L)] for q in range(NQ)]
        iota = lax.iota(jnp.int32, L)

        def gather(g, b):
            return pltpu.make_async_copy(
                table_hbm.at[pair_v.at[g]], ibuf[b], gsem[b])

        def out_copy(g, b):
            return pltpu.make_async_copy(
                obuf[b], out_hbm.at[pl.ds(base + g * GROUP, GROUP)],
                osem[b])

        def compute_group(g, b):
            src, dst = ibuf[b], obuf[b]

            def row_body(r):
                rb = lax.bitwise_and(r, jnp.int32(-16))
                pv = par_v[g, pl.ds(rb, L)]
                pr = pv.at[jnp.full((L,), lax.bitwise_and(r, 15),
                                    jnp.int32)].get(
                                        mode="promise_in_bounds")
                m = pr > 0
                x = [jnp.where(m,
                               src[r, pl.ds(HIDDEN + q * L, L)],
                               src[r, pl.ds(q * L, L)])
                     for q in range(NQ)]
                p = (x[0] + x[1]) + (x[2] + x[3])
                sq = (x[0] * x[0] + x[1] * x[1]) + (x[2] * x[2]
                                                    + x[3] * x[3])
                total = _bsum(p, iota)
                totsq = _bsum(sq, iota)
                mean = total * (1.0 / HIDDEN)
                var = totsq * (1.0 / HIDDEN) - mean * mean
                rstd = _rsqrt(var + EPS)
                nmr = -mean * rstd
                for q in range(NQ):
                    dst[r, pl.ds(q * L, L)] = (
                        (x[q] * rstd + nmr) * g4[q] + b4[q])

            plsc.parallel_loop(0, GROUP, 1, unroll=8)(row_body)

        gather(0, 0).start()
        gather(1, 1).start()

        def group_body(g, _):
            for phase in range(2):
                gg = g * 2 + phase
                pl.when(gg >= 2)(lambda: out_copy(gg - 2, phase).wait())
                gather(gg, phase).wait()
                compute_group(gg, phase)
                out_copy(gg, phase).start()
                pl.when(gg + 2 < ngroups)(
                    lambda: gather(gg + 2, phase).start())
            return 0

        lax.fori_loop(0, ngroups // 2, group_body, 0)
        out_copy(ngroups - 2, 0).wait()
        out_copy(ngroups - 1, 1).wait()

    return sc_kernel


def kernel(input_idx, table, ln_gamma, ln_beta):
    nb, nt = input_idx.shape
    B = nb * nt
    idx = input_idx.reshape(B // GROUP, GROUP).astype(jnp.int32)
    pair = idx >> 1
    par = idx & 1
    table128 = table.reshape(table.shape[0] // 2, 2 * HIDDEN)
    out = _make_sc_kernel(B)(pair, par, table128, ln_gamma, ln_beta)
    return out.reshape(nb, nt, HIDDEN)


# speedup vs baseline: 2.3351x; 1.0588x over previous
import functools

import jax
import jax.numpy as jnp
from jax import lax
from jax.experimental import pallas as pl
from jax.experimental.pallas import tpu as pltpu
from jax.experimental.pallas import tpu_sc as plsc

HIDDEN = 64
NQ = HIDDEN // 16
EPS = 1e-5
L = 16
NC, NS = 2, 16
NW = NC * NS
GROUP = 128
PADW = 128


def _rsqrt(x):
    i = lax.bitcast_convert_type(x, jnp.int32)
    i = jnp.int32(0x5F3759DF) - (i >> 1)
    y = lax.bitcast_convert_type(i, jnp.float32)
    for _ in range(2):
        y = y * (1.5 - 0.5 * x * y * y)
    return y


def _bsum(v, iota):
    for sh in (8, 4, 2, 1):
        v = v + v.at[iota ^ sh].get(mode="promise_in_bounds")
    return v


def _make_sc_kernel(B):
    per_tile = B // NW
    ngroups = per_tile // GROUP
    mesh = plsc.VectorSubcoreMesh(
        core_axis_name="c", subcore_axis_name="s",
        num_cores=NC, num_subcores=NS)

    @functools.partial(
        pl.kernel,
        out_type=jax.ShapeDtypeStruct((B, HIDDEN), jnp.float32),
        mesh=mesh,
        scratch_types=[
            pltpu.VMEM((per_tile // GROUP, GROUP), jnp.int32),
            pltpu.VMEM((GROUP, PADW), jnp.float32),
            pltpu.VMEM((GROUP, PADW), jnp.float32),
            pltpu.VMEM((GROUP, HIDDEN), jnp.float32),
            pltpu.VMEM((GROUP, HIDDEN), jnp.float32),
            pltpu.VMEM((HIDDEN,), jnp.float32),
            pltpu.VMEM((HIDDEN,), jnp.float32),
            pltpu.SemaphoreType.DMA, pltpu.SemaphoreType.DMA,
            pltpu.SemaphoreType.DMA, pltpu.SemaphoreType.DMA,
        ],
        compiler_params=pltpu.CompilerParams(
            needs_layout_passes=False, use_tc_tiling_on_sc=True),
    )
    def sc_kernel(idx_hbm, table_hbm, gamma_hbm, beta_hbm, out_hbm,
                  idx_v, ibuf0, ibuf1, obuf0, obuf1,
                  gamma_v, beta_v, gsem0, gsem1, osem0, osem1):
        ibuf = (ibuf0, ibuf1)
        obuf = (obuf0, obuf1)
        gsem = (gsem0, gsem1)
        osem = (osem0, osem1)
        wid = lax.axis_index("s") * NC + lax.axis_index("c")
        base = wid * per_tile

        pltpu.sync_copy(idx_hbm.at[pl.ds(wid * ngroups, ngroups)], idx_v)
        pltpu.sync_copy(gamma_hbm, gamma_v)
        pltpu.sync_copy(beta_hbm, beta_v)

        g4 = [gamma_v[pl.ds(q * L, L)] for q in range(NQ)]
        b4 = [beta_v[pl.ds(q * L, L)] for q in range(NQ)]
        iota = lax.iota(jnp.int32, L)

        def gather(g, b):
            return pltpu.make_async_copy(
                table_hbm.at[idx_v.at[g]], ibuf[b], gsem[b])

        def out_copy(g, b):
            return pltpu.make_async_copy(
                obuf[b], out_hbm.at[pl.ds(base + g * GROUP, GROUP)],
                osem[b])

        def compute_group(b):
            src, dst = ibuf[b], obuf[b]

            def row_body(r):
                x = [src[r, pl.ds(q * L, L)] for q in range(NQ)]
                p = (x[0] + x[1]) + (x[2] + x[3])
                sq = (x[0] * x[0] + x[1] * x[1]) + (x[2] * x[2]
                                                    + x[3] * x[3])
                total = _bsum(p, iota)
                totsq = _bsum(sq, iota)
                mean = total * (1.0 / HIDDEN)
                var = totsq * (1.0 / HIDDEN) - mean * mean
                rstd = _rsqrt(var + EPS)
                nmr = -mean * rstd
                for q in range(NQ):
                    dst[r, pl.ds(q * L, L)] = (
                        (x[q] * rstd + nmr) * g4[q] + b4[q])

            plsc.parallel_loop(0, GROUP, 1, unroll=8)(row_body)

        gather(0, 0).start()
        gather(1, 1).start()

        def group_body(g, _):
            for phase in range(2):
                gg = g * 2 + phase
                pl.when(gg >= 2)(lambda: out_copy(gg - 2, phase).wait())
                gather(gg, phase).wait()
                compute_group(phase)
                out_copy(gg, phase).start()
                pl.when(gg + 2 < ngroups)(
                    lambda: gather(gg + 2, phase).start())
            return 0

        lax.fori_loop(0, ngroups // 2, group_body, 0)
        out_copy(ngroups - 2, 0).wait()
        out_copy(ngroups - 1, 1).wait()

    return sc_kernel


def kernel(input_idx, table, ln_gamma, ln_beta):
    nb, nt = input_idx.shape
    B = nb * nt
    idx = input_idx.reshape(B // GROUP, GROUP).astype(jnp.int32)
    table_pad = jnp.pad(table, ((0, 0), (0, PADW - HIDDEN)))
    out = _make_sc_kernel(B)(idx, table_pad, ln_gamma, ln_beta)
    return out.reshape(nb, nt, HIDDEN)


# speedup vs baseline: 2.7642x; 1.1838x over previous
import functools

import jax
import jax.numpy as jnp
from jax import lax
from jax.experimental import layout as jlayout
from jax.experimental import pallas as pl
from jax.experimental.pallas import tpu as pltpu
from jax.experimental.pallas import tpu_sc as plsc

HIDDEN = 64
NQ = HIDDEN // 16
EPS = 1e-5
L = 16
NC, NS = 2, 16
NW = NC * NS
GROUP = 128
PADW = 128


def _rsqrt(x):
    i = lax.bitcast_convert_type(x, jnp.int32)
    i = jnp.int32(0x5F3759DF) - (i >> 1)
    y = lax.bitcast_convert_type(i, jnp.float32)
    for _ in range(2):
        y = y * (1.5 - 0.5 * x * y * y)
    return y


def _bsum(v, iota):
    for sh in (8, 4, 2, 1):
        v = v + v.at[iota ^ sh].get(mode="promise_in_bounds")
    return v


def _make_sc_kernel(B):
    per_tile = B // NW
    ngroups = per_tile // GROUP
    mesh = plsc.VectorSubcoreMesh(
        core_axis_name="c", subcore_axis_name="s",
        num_cores=NC, num_subcores=NS)

    @functools.partial(
        pl.kernel,
        out_type=jax.ShapeDtypeStruct((B, HIDDEN), jnp.float32),
        mesh=mesh,
        scratch_types=[
            pltpu.VMEM((per_tile // GROUP, GROUP), jnp.int32),
            pltpu.VMEM((GROUP, PADW), jnp.float32),
            pltpu.VMEM((GROUP, PADW), jnp.float32),
            pltpu.VMEM((GROUP, HIDDEN), jnp.float32),
            pltpu.VMEM((GROUP, HIDDEN), jnp.float32),
            pltpu.VMEM((HIDDEN,), jnp.float32),
            pltpu.VMEM((HIDDEN,), jnp.float32),
            pltpu.SemaphoreType.DMA, pltpu.SemaphoreType.DMA,
            pltpu.SemaphoreType.DMA, pltpu.SemaphoreType.DMA,
        ],
        compiler_params=pltpu.CompilerParams(
            needs_layout_passes=False, use_tc_tiling_on_sc=True),
    )
    def sc_kernel(idx_hbm, table_hbm, gamma_hbm, beta_hbm, out_hbm,
                  idx_v, ibuf0, ibuf1, obuf0, obuf1,
                  gamma_v, beta_v, gsem0, gsem1, osem0, osem1):
        ibuf = (ibuf0, ibuf1)
        obuf = (obuf0, obuf1)
        gsem = (gsem0, gsem1)
        osem = (osem0, osem1)
        wid = lax.axis_index("s") * NC + lax.axis_index("c")
        base = wid * per_tile

        pltpu.sync_copy(idx_hbm.at[pl.ds(wid * ngroups, ngroups)], idx_v)
        pltpu.sync_copy(gamma_hbm, gamma_v)
        pltpu.sync_copy(beta_hbm, beta_v)

        g4 = [gamma_v[pl.ds(q * L, L)] for q in range(NQ)]
        b4 = [beta_v[pl.ds(q * L, L)] for q in range(NQ)]
        iota = lax.iota(jnp.int32, L)

        def gather(g, b):
            return pltpu.make_async_copy(
                table_hbm.at[idx_v.at[g]], ibuf[b], gsem[b])

        def out_copy(g, b):
            return pltpu.make_async_copy(
                obuf[b], out_hbm.at[pl.ds(base + g * GROUP, GROUP)],
                osem[b])

        def compute_group(b):
            src, dst = ibuf[b], obuf[b]

            def row_body(r):
                x = [src[r, pl.ds(q * L, L)] for q in range(NQ)]
                p = (x[0] + x[1]) + (x[2] + x[3])
                sq = (x[0] * x[0] + x[1] * x[1]) + (x[2] * x[2]
                                                    + x[3] * x[3])
                total = _bsum(p, iota)
                totsq = _bsum(sq, iota)
                mean = total * (1.0 / HIDDEN)
                var = totsq * (1.0 / HIDDEN) - mean * mean
                rstd = _rsqrt(var + EPS)
                nmr = -mean * rstd
                for q in range(NQ):
                    dst[r, pl.ds(q * L, L)] = (
                        (x[q] * rstd + nmr) * g4[q] + b4[q])

            plsc.parallel_loop(0, GROUP, 1, unroll=8)(row_body)

        gather(0, 0).start()
        gather(1, 1).start()

        def group_body(g, _):
            for phase in range(2):
                gg = g * 2 + phase
                pl.when(gg >= 2)(lambda: out_copy(gg - 2, phase).wait())
                gather(gg, phase).wait()
                compute_group(phase)
                out_copy(gg, phase).start()
                pl.when(gg + 2 < ngroups)(
                    lambda: gather(gg + 2, phase).start())
            return 0

        lax.fori_loop(0, ngroups // 2, group_body, 0)
        out_copy(ngroups - 2, 0).wait()
        out_copy(ngroups - 1, 1).wait()

    return sc_kernel


def kernel(input_idx, table, ln_gamma, ln_beta):
    nb, nt = input_idx.shape
    B = nb * nt
    idx = input_idx.reshape(B // GROUP, GROUP).astype(jnp.int32)
    table_pad = jnp.concatenate(
        [table, jnp.zeros_like(table)], axis=1)
    out = _make_sc_kernel(B)(idx, table_pad, ln_gamma, ln_beta)
    out3 = out.reshape(nb, nt, HIDDEN)
    return jlayout.with_layout_constraint(
        out3, jlayout.Layout(major_to_minor=(0, 1, 2)))
